# Initial kernel scaffold; baseline (speedup 1.0000x reference)
import jax
import jax.numpy as jnp
from jax.experimental import pallas as pl


def kernel(x, edge_index, batch, W1, b1, W2, b2, W3, b3, Wl, bl):
    raise NotImplementedError("write your pallas kernel here")


# speedup vs baseline: 2.2450x; 2.2450x over previous
import functools

import jax
import jax.numpy as jnp
from jax import lax
from jax.experimental import pallas as pl
from jax.experimental.pallas import tpu as pltpu

N_NODES = 10000
N_GRAPHS = 64
RB = 1000
GRID = N_NODES // RB


def _u1_body(deg_ref, x_ref, wt_ref, o_ref):
    dinv = lax.rsqrt(deg_ref[...] + 1.0)
    o_ref[...] = dinv * jnp.dot(x_ref[...], wt_ref[...],
                                preferred_element_type=jnp.float32)


def _u1(deg, x, wt):
    f_in, f_out = wt.shape
    return pl.pallas_call(
        _u1_body,
        grid=(GRID,),
        in_specs=[
            pl.BlockSpec((RB, 1), lambda i: (i, 0)),
            pl.BlockSpec((RB, f_in), lambda i: (i, 0)),
            pl.BlockSpec((f_in, f_out), lambda i: (0, 0)),
        ],
        out_specs=pl.BlockSpec((RB, f_out), lambda i: (i, 0)),
        out_shape=jax.ShapeDtypeStruct((N_NODES, f_out), jnp.float32),
    )(deg, x, wt)


def _umid_body(deg_ref, agg_ref, u_ref, b_ref, wt_ref, o_ref):
    dinv = lax.rsqrt(deg_ref[...] + 1.0)
    h = jnp.maximum(dinv * (agg_ref[...] + u_ref[...]) + b_ref[...], 0.0)
    o_ref[...] = dinv * jnp.dot(h, wt_ref[...],
                                preferred_element_type=jnp.float32)


def _umid(deg, agg, u, b, wt):
    f_in, f_out = wt.shape
    return pl.pallas_call(
        _umid_body,
        grid=(GRID,),
        in_specs=[
            pl.BlockSpec((RB, 1), lambda i: (i, 0)),
            pl.BlockSpec((RB, f_in), lambda i: (i, 0)),
            pl.BlockSpec((RB, f_in), lambda i: (i, 0)),
            pl.BlockSpec((1, f_in), lambda i: (0, 0)),
            pl.BlockSpec((f_in, f_out), lambda i: (0, 0)),
        ],
        out_specs=pl.BlockSpec((RB, f_out), lambda i: (i, 0)),
        out_shape=jax.ShapeDtypeStruct((N_NODES, f_out), jnp.float32),
    )(deg, agg, u, b, wt)


def _pool_body(deg_ref, agg_ref, u_ref, b_ref, batch_ref, wt_ref, bl_ref,
               o_ref, sums_ref, cnts_ref):
    i = pl.program_id(0)

    @pl.when(i == 0)
    def _init():
        sums_ref[...] = jnp.zeros_like(sums_ref)
        cnts_ref[...] = jnp.zeros_like(cnts_ref)

    dinv = lax.rsqrt(deg_ref[...] + 1.0)
    h = dinv * (agg_ref[...] + u_ref[...]) + b_ref[...]
    gids = lax.broadcasted_iota(jnp.int32, (N_GRAPHS, RB), 0)
    onehot = (batch_ref[0] == gids).astype(jnp.float32)
    sums_ref[...] += jnp.dot(onehot, h, preferred_element_type=jnp.float32)
    cnts_ref[...] += jnp.sum(onehot, axis=1, keepdims=True)

    @pl.when(i == GRID - 1)
    def _fin():
        g = sums_ref[...] / jnp.maximum(cnts_ref[...], 1.0)
        o_ref[...] = jnp.dot(g, wt_ref[...],
                             preferred_element_type=jnp.float32) + bl_ref[...]


def _pool(deg, agg, u, b, batch3, wt, bl):
    f_in, f_out = wt.shape
    return pl.pallas_call(
        _pool_body,
        grid=(GRID,),
        in_specs=[
            pl.BlockSpec((RB, 1), lambda i: (i, 0)),
            pl.BlockSpec((RB, f_in), lambda i: (i, 0)),
            pl.BlockSpec((RB, f_in), lambda i: (i, 0)),
            pl.BlockSpec((1, f_in), lambda i: (0, 0)),
            pl.BlockSpec((1, 1, RB), lambda i: (i, 0, 0)),
            pl.BlockSpec((f_in, f_out), lambda i: (0, 0)),
            pl.BlockSpec((1, f_out), lambda i: (0, 0)),
        ],
        out_specs=pl.BlockSpec((N_GRAPHS, f_out), lambda i: (0, 0)),
        out_shape=jax.ShapeDtypeStruct((N_GRAPHS, f_out), jnp.float32),
        scratch_shapes=[
            pltpu.VMEM((N_GRAPHS, f_in), jnp.float32),
            pltpu.VMEM((N_GRAPHS, 1), jnp.float32),
        ],
    )(deg, agg, u, b, batch3, wt, bl)


def kernel(x, edge_index, batch, W1, b1, W2, b2, W3, b3, Wl, bl):
    src = edge_index[0].astype(jnp.int32)
    dst = edge_index[1].astype(jnp.int32)

    deg = jnp.zeros((N_NODES,), jnp.float32).at[dst].add(1.0)

    def agg(u):
        return jnp.zeros_like(u).at[dst].add(u[src])

    deg2 = deg.reshape(N_NODES, 1)
    batch3 = batch.astype(jnp.int32).reshape(GRID, 1, RB)

    u1 = _u1(deg2, x, W1.T)
    a1 = agg(u1)
    u2 = _umid(deg2, a1, u1, b1.reshape(1, -1), W2.T)
    a2 = agg(u2)
    u3 = _umid(deg2, a2, u2, b2.reshape(1, -1), W3.T)
    a3 = agg(u3)
    return _pool(deg2, a3, u3, b3.reshape(1, -1), batch3, Wl.T,
                 bl.reshape(1, -1))


# speedup vs baseline: 2.2459x; 1.0004x over previous
import functools

import jax
import jax.numpy as jnp
from jax import lax
from jax.experimental import pallas as pl
from jax.experimental.pallas import tpu as pltpu
from jax.experimental.pallas import tpu_sc as plsc

N_NODES = 10000
N_GRAPHS = 64
N_EDGES = 160000
F = 512

RB = 1000
GRID = N_NODES // RB

NC, NS = 2, 16
EROWS = 79
EPS = EROWS * 128
EPAD = NS * EPS
HROWS = 10016
SLAB = 2500
SLABP = 2528
LROWS = 80

_MESH = plsc.VectorSubcoreMesh(core_axis_name="c", subcore_axis_name="s")


def _zero_vmem(ref, rows, width):
    def body(r, _):
        for k in range(width // 16):
            ref[r, pl.ds(k * 16, 16)] = jnp.zeros((16,), ref.dtype)
        return 0

    lax.fori_loop(0, rows, body, 0)


def _sc_prep_body(src5_h, dst5_h,
                  degs_h, srcL_h, dstL_h, cnts_h,
                  svm, dvm, ones, zb, sq0, sq1, dq0, dq1,
                  cbuf, cnt0, cnt1, slab):
    c = lax.axis_index("c")
    s = lax.axis_index("s")

    pltpu.sync_copy(src5_h.at[s], svm)
    pltpu.sync_copy(dst5_h.at[s], dvm)

    _zero_vmem(zb, HROWS // NS, 16)
    pltpu.sync_copy(zb, slab.at[pl.ds(s * (HROWS // NS), HROWS // NS)])

    def ones_body(r, _):
        ones[r, pl.ds(0, 16)] = jnp.ones((16,), jnp.float32)
        return 0

    lax.fori_loop(0, 128, ones_body, 0)
    plsc.subcore_barrier()

    def deg_body(j, _):
        pltpu.sync_copy(ones, slab.at[dvm.at[j]], add=True)
        return 0

    lax.fori_loop(0, EROWS, deg_body, 0)

    base0 = c * (2 * SLAB)
    iota = lax.iota(jnp.int32, 16)
    lane15 = jnp.full((16,), 15, jnp.int32)
    zero16 = jnp.zeros((16,), jnp.int32)
    cnt0[pl.ds(0, 16)] = zero16
    cnt1[pl.ds(0, 16)] = zero16

    def comp_body(i, _):
        for k in range(8):
            s16 = svm[i, pl.ds(k * 16, 16)]
            d16 = dvm[i, pl.ds(k * 16, 16)]
            lo0 = d16 - base0
            m0 = (lo0 >= 0) & (lo0 < SLAB)
            lo1 = lo0 - SLAB
            m1 = (lo1 >= 0) & (lo1 < SLAB)
            c0v = cnt0[pl.ds(0, 16)]
            inc0 = plsc.cumsum(m0.astype(jnp.int32))
            pos0 = c0v + inc0 - 1
            plsc.store_scatter(sq0, [pos0 >> 7, pos0 & 127], s16, mask=m0)
            plsc.store_scatter(dq0, [pos0 >> 7, pos0 & 127], lo0, mask=m0)
            cbuf[pl.ds(0, 16)] = inc0
            cnt0[pl.ds(0, 16)] = c0v + plsc.load_gather(cbuf, [lane15])
            c1v = cnt1[pl.ds(0, 16)]
            inc1 = plsc.cumsum(m1.astype(jnp.int32))
            pos1 = c1v + inc1 - 1
            plsc.store_scatter(sq1, [pos1 >> 7, pos1 & 127], s16, mask=m1)
            plsc.store_scatter(dq1, [pos1 >> 7, pos1 & 127], lo1, mask=m1)
            cbuf[pl.ds(0, 16)] = inc1
            cnt1[pl.ds(0, 16)] = c1v + plsc.load_gather(cbuf, [lane15])
        return 0

    lax.fori_loop(0, EROWS, comp_body, 0)

    c0v = cnt0[pl.ds(0, 16)]
    c1v = cnt1[pl.ds(0, 16)]
    pad_d = jnp.full((16,), SLAB + s, jnp.int32)
    for k in range(8):
        p0 = c0v + iota + 16 * k
        plsc.store_scatter(sq0, [p0 >> 7, p0 & 127], iota, mask=iota < 16)
        plsc.store_scatter(dq0, [p0 >> 7, p0 & 127], pad_d, mask=iota < 16)
        p1 = c1v + iota + 16 * k
        plsc.store_scatter(sq1, [p1 >> 7, p1 & 127], iota, mask=iota < 16)
        plsc.store_scatter(dq1, [p1 >> 7, p1 & 127], pad_d, mask=iota < 16)

    for q, (sq, dq, cqv) in enumerate(((sq0, dq0, c0v), (sq1, dq1, c1v))):
        pltpu.sync_copy(sq, srcL_h.at[c, q, s])
        pltpu.sync_copy(dq, dstL_h.at[c, q, s])
        cbuf[pl.ds(0, 16)] = cqv
        pltpu.sync_copy(cbuf, cnts_h.at[c, q, s])

    plsc.subcore_barrier()

    @pl.when(s == 0)
    def _():
        pltpu.sync_copy(slab, degs_h.at[c])


def _sc_prep(src5, dst5):
    return pl.kernel(
        _sc_prep_body,
        out_type=[
            jax.ShapeDtypeStruct((NC, HROWS, 16), jnp.float32),
            jax.ShapeDtypeStruct((NC, 2, NS, LROWS, 128), jnp.int32),
            jax.ShapeDtypeStruct((NC, 2, NS, LROWS, 128), jnp.int32),
            jax.ShapeDtypeStruct((NC, 2, NS, 16), jnp.int32),
        ],
        mesh=_MESH,
        scratch_types=[
            pltpu.VMEM((EROWS, 128), jnp.int32),
            pltpu.VMEM((EROWS, 128), jnp.int32),
            pltpu.VMEM((128, 16), jnp.float32),
            pltpu.VMEM((HROWS // NS, 16), jnp.float32),
            pltpu.VMEM((LROWS, 128), jnp.int32),
            pltpu.VMEM((LROWS, 128), jnp.int32),
            pltpu.VMEM((LROWS, 128), jnp.int32),
            pltpu.VMEM((LROWS, 128), jnp.int32),
            pltpu.VMEM((16,), jnp.int32),
            pltpu.VMEM((16,), jnp.int32),
            pltpu.VMEM((16,), jnp.int32),
            pltpu.VMEM_SHARED((HROWS, 16), jnp.float32),
        ],
    )(src5, dst5)


def _sc_agg_body(u_h, srcL_h, dstL_h, cnts_h,
                 out_h,
                 srcs, d2d, rows, zb, cvec, slab, sem):
    c = lax.axis_index("c")
    s = lax.axis_index("s")
    stripe = SLABP // NS

    _zero_vmem(zb, stripe // 2, F)

    for q in range(2):
        pltpu.sync_copy(zb, slab.at[pl.ds(s * stripe, stripe // 2)])
        pltpu.sync_copy(zb, slab.at[pl.ds(s * stripe + stripe // 2,
                                          stripe // 2)])
        plsc.subcore_barrier()

        pltpu.sync_copy(srcL_h.at[c, q, s], srcs)
        pltpu.sync_copy(dstL_h.at[c, q, s], d2d)
        pltpu.sync_copy(cnts_h.at[c, q, s], cvec)
        cv = cvec[pl.ds(0, 16)]
        cnt = cv[0]
        nb = (cnt + 127) // 128

        def batch_body(b, _):
            pltpu.async_copy(u_h.at[srcs.at[b]], rows, sem).wait()
            pltpu.sync_copy(rows, slab.at[d2d.at[b]], add=True)
            return 0

        lax.fori_loop(0, nb, batch_body, 0)
        plsc.subcore_barrier()

        base = (2 * c + q) * SLAB

        @pl.when(s < NS - 1)
        def _full():
            pltpu.sync_copy(slab.at[pl.ds(s * stripe, stripe)],
                            out_h.at[pl.ds(base + s * stripe, stripe)])

        @pl.when(s == NS - 1)
        def _clip():
            last = SLAB - (NS - 1) * stripe
            pltpu.sync_copy(slab.at[pl.ds((NS - 1) * stripe, last)],
                            out_h.at[pl.ds(base + (NS - 1) * stripe, last)])

        plsc.subcore_barrier()


def _sc_agg(u, srcL, dstL, cnts):
    return pl.kernel(
        _sc_agg_body,
        out_type=jax.ShapeDtypeStruct((N_NODES, F), jnp.float32),
        mesh=_MESH,
        scratch_types=[
            pltpu.VMEM((LROWS, 128), jnp.int32),
            pltpu.VMEM((LROWS, 128), jnp.int32),
            pltpu.VMEM((128, F), jnp.float32),
            pltpu.VMEM((SLABP // NS // 2, F), jnp.float32),
            pltpu.VMEM((16,), jnp.int32),
            pltpu.VMEM_SHARED((SLABP, F), jnp.float32),
            pltpu.SemaphoreType.DMA,
        ],
    )(u, srcL, dstL, cnts)


def _dinv(deg_ref):
    return lax.rsqrt(deg_ref[0, :, 0:1] + 1.0)


def _u1_body(deg_ref, x_ref, wt_ref, o_ref):
    o_ref[...] = _dinv(deg_ref) * jnp.dot(x_ref[...], wt_ref[...],
                                          preferred_element_type=jnp.float32)


def _u1(degs, x, wt):
    f_in, f_out = wt.shape
    return pl.pallas_call(
        _u1_body,
        grid=(GRID,),
        in_specs=[
            pl.BlockSpec((1, RB, 16), lambda i: (0, i, 0)),
            pl.BlockSpec((RB, f_in), lambda i: (i, 0)),
            pl.BlockSpec((f_in, f_out), lambda i: (0, 0)),
        ],
        out_specs=pl.BlockSpec((RB, f_out), lambda i: (i, 0)),
        out_shape=jax.ShapeDtypeStruct((N_NODES, f_out), jnp.float32),
    )(degs, x, wt)


def _umid_body(deg_ref, agg_ref, u_ref, b_ref, wt_ref, o_ref):
    dinv = _dinv(deg_ref)
    h = jnp.maximum(dinv * (agg_ref[...] + u_ref[...]) + b_ref[...], 0.0)
    o_ref[...] = dinv * jnp.dot(h, wt_ref[...],
                                preferred_element_type=jnp.float32)


def _umid(degs, agg, u, b, wt):
    f_in, f_out = wt.shape
    return pl.pallas_call(
        _umid_body,
        grid=(GRID,),
        in_specs=[
            pl.BlockSpec((1, RB, 16), lambda i: (0, i, 0)),
            pl.BlockSpec((RB, f_in), lambda i: (i, 0)),
            pl.BlockSpec((RB, f_in), lambda i: (i, 0)),
            pl.BlockSpec((1, f_in), lambda i: (0, 0)),
            pl.BlockSpec((f_in, f_out), lambda i: (0, 0)),
        ],
        out_specs=pl.BlockSpec((RB, f_out), lambda i: (i, 0)),
        out_shape=jax.ShapeDtypeStruct((N_NODES, f_out), jnp.float32),
    )(degs, agg, u, b, wt)


def _pool_body(deg_ref, agg_ref, u_ref, b_ref, batch_ref, wt_ref, bl_ref,
               o_ref, sums_ref, cnts_ref):
    i = pl.program_id(0)

    @pl.when(i == 0)
    def _init():
        sums_ref[...] = jnp.zeros_like(sums_ref)
        cnts_ref[...] = jnp.zeros_like(cnts_ref)

    h = _dinv(deg_ref) * (agg_ref[...] + u_ref[...]) + b_ref[...]
    gids = lax.broadcasted_iota(jnp.int32, (N_GRAPHS, RB), 0)
    onehot = (batch_ref[0] == gids).astype(jnp.float32)
    sums_ref[...] += jnp.dot(onehot, h, preferred_element_type=jnp.float32)
    cnts_ref[...] += jnp.sum(onehot, axis=1, keepdims=True)

    @pl.when(i == GRID - 1)
    def _fin():
        g = sums_ref[...] / jnp.maximum(cnts_ref[...], 1.0)
        o_ref[...] = jnp.dot(g, wt_ref[...],
                             preferred_element_type=jnp.float32) + bl_ref[...]


def _pool(degs, agg, u, b, batch3, wt, bl):
    f_in, f_out = wt.shape
    return pl.pallas_call(
        _pool_body,
        grid=(GRID,),
        in_specs=[
            pl.BlockSpec((1, RB, 16), lambda i: (0, i, 0)),
            pl.BlockSpec((RB, f_in), lambda i: (i, 0)),
            pl.BlockSpec((RB, f_in), lambda i: (i, 0)),
            pl.BlockSpec((1, f_in), lambda i: (0, 0)),
            pl.BlockSpec((1, 1, RB), lambda i: (i, 0, 0)),
            pl.BlockSpec((f_in, f_out), lambda i: (0, 0)),
            pl.BlockSpec((1, f_out), lambda i: (0, 0)),
        ],
        out_specs=pl.BlockSpec((N_GRAPHS, f_out), lambda i: (0, 0)),
        out_shape=jax.ShapeDtypeStruct((N_GRAPHS, f_out), jnp.float32),
        scratch_shapes=[
            pltpu.VMEM((N_GRAPHS, f_in), jnp.float32),
            pltpu.VMEM((N_GRAPHS, 1), jnp.float32),
        ],
    )(degs, agg, u, b, batch3, wt, bl)


def kernel(x, edge_index, batch, W1, b1, W2, b2, W3, b3, Wl, bl):
    src = edge_index[0].astype(jnp.int32)
    dst = edge_index[1].astype(jnp.int32)
    batch3 = batch.astype(jnp.int32).reshape(GRID, 1, RB)

    deg = jnp.zeros((N_NODES,), jnp.float32).at[dst].add(1.0)
    degp = jnp.concatenate([deg, jnp.zeros((HROWS - N_NODES,), jnp.float32)])
    degs = jnp.broadcast_to(degp[None, :, None], (1, HROWS, 16))

    def _jagg(u):
        return jnp.zeros_like(u).at[dst].add(u[src])

    u1 = _u1(degs, x, W1.T)
    a1 = _jagg(u1)
    u2 = _umid(degs, a1, u1, b1.reshape(1, -1), W2.T)
    a2 = _jagg(u2)
    u3 = _umid(degs, a2, u2, b2.reshape(1, -1), W3.T)
    a3 = _jagg(u3)
    return _pool(degs, a3, u3, b3.reshape(1, -1), batch3, Wl.T,
                 bl.reshape(1, -1))
